# baseline (device time: 46096 ns/iter reference)
import jax
import jax.numpy as jnp
from jax import lax
from jax.experimental import pallas as pl
from jax.experimental.pallas import tpu as pltpu

N_DEV = 4
DH = 128
SCALE = 0.08838834764831843


def kernel(x, Wq, Wo, Wk, Wv):
    _, sq, d = x.shape
    d_local = Wq.shape[1]
    n_heads = d_local // DH

    def body(x_ref, wq_ref, wo_ref, wk_ref, wv_ref, out_ref,
             xloc, xg, wqkv, wob, obuf, pstage, precv,
             xs_sems, xr_sems, s_sems, r_sems):
        my = lax.axis_index("i")
        left = lax.rem(my + (N_DEV - 1), N_DEV)
        right = lax.rem(my + 1, N_DEV)
        diag = lax.rem(my + 2, N_DEV)

        barrier_sem = pltpu.get_barrier_semaphore()
        for nbr in (left, right, diag):
            pl.semaphore_signal(barrier_sem, inc=1, device_id=(nbr,),
                                device_id_type=pl.DeviceIdType.MESH)
        pl.semaphore_wait(barrier_sem, 3)

        def rdma(src, dst, sem_s, sem_r, idx, dev):
            return pltpu.make_async_remote_copy(
                src_ref=src, dst_ref=dst,
                send_sem=sem_s.at[idx], recv_sem=sem_r.at[idx],
                device_id=(dev,), device_id_type=pl.DeviceIdType.MESH,
            )

        xloc[...] = x_ref[0].astype(jnp.bfloat16)

        a_r = rdma(xloc, xg.at[0], xs_sems, xr_sems, 0, right)
        a_l = rdma(xloc, xg.at[1], xs_sems, xr_sems, 1, left)
        a_x = rdma(xloc, xg.at[2], xs_sems, xr_sems, 2, diag)
        r_r = rdma(pstage.at[0], precv.at[0], s_sems, r_sems, 0, right)
        r_l = rdma(pstage.at[1], precv.at[1], s_sems, r_sems, 1, left)
        r_x = rdma(pstage.at[2], precv.at[2], s_sems, r_sems, 2, diag)

        a_r.start()
        a_l.start()
        a_x.start()

        wqkv[:, pl.ds(0, d_local)] = (wq_ref[...] * SCALE).astype(jnp.bfloat16)
        wqkv[:, pl.ds(d_local, d_local)] = wk_ref[...].astype(jnp.bfloat16)
        wqkv[:, pl.ds(2 * d_local, d_local)] = wv_ref[...].astype(jnp.bfloat16)
        wob[...] = wo_ref[...].astype(jnp.bfloat16)

        def compute_partial(xb):
            qkv = jnp.dot(xb, wqkv[...],
                          preferred_element_type=jnp.float32
                          ).astype(jnp.bfloat16)
            for hh in range(n_heads):
                q = qkv[:, hh * DH:(hh + 1) * DH]
                k = qkv[:, d_local + hh * DH:d_local + (hh + 1) * DH]
                v = qkv[:, 2 * d_local + hh * DH:2 * d_local + (hh + 1) * DH]
                s = lax.dot_general(
                    q, k, (((1,), (1,)), ((), ())),
                    preferred_element_type=jnp.float32,
                )
                p = jnp.exp(s)
                l = jnp.sum(p, axis=1, keepdims=True)
                o = jnp.dot(p.astype(jnp.bfloat16), v,
                            preferred_element_type=jnp.float32) / l
                obuf[:, hh * DH:(hh + 1) * DH] = o.astype(jnp.bfloat16)
            return jnp.dot(obuf[...], wob[...],
                           preferred_element_type=jnp.float32)

        a_x.wait_recv()
        pstage[2] = compute_partial(xg[2]).astype(jnp.bfloat16)
        r_x.start()

        a_l.wait_recv()
        pstage[0] = compute_partial(xg[1]).astype(jnp.bfloat16)
        r_r.start()

        a_r.wait_recv()
        pstage[1] = compute_partial(xg[0]).astype(jnp.bfloat16)
        r_l.start()

        p0 = compute_partial(xloc[...])

        r_r.wait_recv()
        r_l.wait_recv()
        r_x.wait_recv()
        out_ref[0] = (p0
                      + precv[0].astype(jnp.float32)
                      + precv[1].astype(jnp.float32)
                      + precv[2].astype(jnp.float32))

        for r in (a_r, a_l, a_x, r_r, r_l, r_x):
            r.wait_send()

    return pl.pallas_call(
        body,
        out_shape=jax.ShapeDtypeStruct((1, sq, d), jnp.float32),
        in_specs=[pl.BlockSpec(memory_space=pltpu.VMEM)] * 5,
        out_specs=pl.BlockSpec(memory_space=pltpu.VMEM),
        scratch_shapes=[
            pltpu.VMEM((sq, d), jnp.bfloat16),
            pltpu.VMEM((3, sq, d), jnp.bfloat16),
            pltpu.VMEM((d, 3 * d_local), jnp.bfloat16),
            pltpu.VMEM((d_local, d), jnp.bfloat16),
            pltpu.VMEM((sq, d_local), jnp.bfloat16),
            pltpu.VMEM((3, sq, d), jnp.bfloat16),
            pltpu.VMEM((3, sq, d), jnp.bfloat16),
            pltpu.SemaphoreType.DMA((3,)),
            pltpu.SemaphoreType.DMA((3,)),
            pltpu.SemaphoreType.DMA((3,)),
            pltpu.SemaphoreType.DMA((3,)),
        ],
        compiler_params=pltpu.CompilerParams(collective_id=0),
    )(x, Wq, Wo, Wk, Wv)


# device time: 43649 ns/iter; 1.0561x vs baseline; 1.0561x over previous
import jax
import jax.numpy as jnp
from jax import lax
from jax.experimental import pallas as pl
from jax.experimental.pallas import tpu as pltpu

N_DEV = 4
DH = 128
SCALE = 0.08838834764831843


def kernel(x, Wq, Wo, Wk, Wv):
    _, sq, d = x.shape
    d_local = Wq.shape[1]
    n_heads = d_local // DH
    hs = sq // 2

    def body(x_ref, wq_ref, wo_ref, wk_ref, wv_ref, out_ref,
             xloc, xg, wqkv, wob, obuf, sfull, shalf, tbuf, dfull, dhalf,
             xs_sems, xr_sems, s_sems, r_sems):
        my = lax.axis_index("i")
        left = lax.rem(my + (N_DEV - 1), N_DEV)
        right = lax.rem(my + 1, N_DEV)

        barrier_sem = pltpu.get_barrier_semaphore()
        for nbr in (left, right):
            pl.semaphore_signal(barrier_sem, inc=1, device_id=(nbr,),
                                device_id_type=pl.DeviceIdType.MESH)
        pl.semaphore_wait(barrier_sem, 2)

        def rdma(src, dst, sem_s, sem_r, idx, dev):
            return pltpu.make_async_remote_copy(
                src_ref=src, dst_ref=dst,
                send_sem=sem_s.at[idx], recv_sem=sem_r.at[idx],
                device_id=(dev,), device_id_type=pl.DeviceIdType.MESH,
            )

        top = pl.ds(0, hs)
        bot = pl.ds(hs, hs)

        xloc[...] = x_ref[0].astype(jnp.bfloat16)

        ag_r0 = rdma(xloc, xg.at[0], xs_sems, xr_sems, 0, right)
        ag_l0 = rdma(xloc, xg.at[1], xs_sems, xr_sems, 1, left)
        ag_r1 = rdma(xg.at[0, top], xg.at[2, top], xs_sems, xr_sems, 2, right)
        ag_l1 = rdma(xg.at[1, bot], xg.at[2, bot], xs_sems, xr_sems, 3, left)
        s_dr_t = rdma(sfull.at[0, top], dfull.at[0, top], s_sems, r_sems,
                      0, right)
        s_dr_b = rdma(sfull.at[0, bot], dfull.at[0, bot], s_sems, r_sems,
                      1, right)
        s_dl_t = rdma(sfull.at[1, top], dfull.at[1, top], s_sems, r_sems,
                      2, left)
        s_dl_b = rdma(sfull.at[1, bot], dfull.at[1, bot], s_sems, r_sems,
                      3, left)
        s_hr = rdma(shalf.at[0], tbuf.at[0], s_sems, r_sems, 4, right)
        s_hl = rdma(shalf.at[1], tbuf.at[1], s_sems, r_sems, 5, left)
        f_r = rdma(tbuf.at[0], dhalf.at[0], s_sems, r_sems, 6, right)
        f_l = rdma(tbuf.at[1], dhalf.at[1], s_sems, r_sems, 7, left)

        ag_r0.start()
        ag_l0.start()

        wqkv[:, pl.ds(0, d_local)] = (wq_ref[...] * SCALE).astype(jnp.bfloat16)
        wqkv[:, pl.ds(d_local, d_local)] = wk_ref[...].astype(jnp.bfloat16)
        wqkv[:, pl.ds(2 * d_local, d_local)] = wv_ref[...].astype(jnp.bfloat16)
        wob[...] = wo_ref[...].astype(jnp.bfloat16)

        def attn_rows(qkv, r0):
            for hh in range(n_heads):
                q = qkv[r0:r0 + hs, hh * DH:(hh + 1) * DH]
                k = qkv[:, d_local + hh * DH:d_local + (hh + 1) * DH]
                v = qkv[:, 2 * d_local + hh * DH:2 * d_local + (hh + 1) * DH]
                s = lax.dot_general(
                    q, k, (((1,), (1,)), ((), ())),
                    preferred_element_type=jnp.float32,
                )
                p = jnp.exp(s)
                l = jnp.sum(p, axis=1, keepdims=True)
                o = jnp.dot(p.astype(jnp.bfloat16), v,
                            preferred_element_type=jnp.float32) / l
                obuf[r0:r0 + hs, hh * DH:(hh + 1) * DH] = o.astype(jnp.bfloat16)
            return jnp.dot(obuf[r0:r0 + hs], wob[...],
                           preferred_element_type=jnp.float32)

        def compute_split(xb, emit_top, emit_bot):
            qkv = jnp.dot(xb, wqkv[...],
                          preferred_element_type=jnp.float32
                          ).astype(jnp.bfloat16)
            emit_top(attn_rows(qkv, 0))
            emit_bot(attn_rows(qkv, hs))

        ag_r0.wait_recv()
        ag_r1.start()
        ag_l0.wait_recv()
        ag_l1.start()

        def emit1_t(v):
            sfull[1, top] = v.astype(jnp.bfloat16)
            s_dl_t.start()

        def emit1_b(v):
            sfull[1, bot] = v.astype(jnp.bfloat16)
            s_dl_b.start()

        compute_split(xg[0], emit1_t, emit1_b)

        ag_r1.wait_recv()
        ag_l1.wait_recv()

        def emit2_t(v):
            shalf[0] = v.astype(jnp.bfloat16)
            s_hr.start()

        def emit2_b(v):
            shalf[1] = v.astype(jnp.bfloat16)
            s_hl.start()

        compute_split(xg[2], emit2_t, emit2_b)

        def emit3_t(v):
            sfull[0, top] = v.astype(jnp.bfloat16)
            s_dr_t.start()
            s_hr.wait_recv()
            f_r.start()
            s_hl.wait_recv()
            f_l.start()

        def emit3_b(v):
            sfull[0, bot] = v.astype(jnp.bfloat16)
            s_dr_b.start()

        compute_split(xg[1], emit3_t, emit3_b)

        qkv0 = jnp.dot(xloc[...], wqkv[...],
                       preferred_element_type=jnp.float32
                       ).astype(jnp.bfloat16)
        p0t = attn_rows(qkv0, 0)
        p0b = attn_rows(qkv0, hs)

        s_dr_t.wait_recv()
        s_dr_b.wait_recv()
        s_dl_t.wait_recv()
        s_dl_b.wait_recv()
        f_r.wait_recv()
        f_l.wait_recv()
        out_ref[0, top] = (p0t
                           + dfull[0, top].astype(jnp.float32)
                           + dfull[1, top].astype(jnp.float32)
                           + dhalf[0].astype(jnp.float32))
        out_ref[0, bot] = (p0b
                           + dfull[0, bot].astype(jnp.float32)
                           + dfull[1, bot].astype(jnp.float32)
                           + dhalf[1].astype(jnp.float32))

        for r in (ag_r0, ag_l0, ag_r1, ag_l1, s_dr_t, s_dr_b, s_dl_t,
                  s_dl_b, s_hr, s_hl, f_r, f_l):
            r.wait_send()

    return pl.pallas_call(
        body,
        out_shape=jax.ShapeDtypeStruct((1, sq, d), jnp.float32),
        in_specs=[pl.BlockSpec(memory_space=pltpu.VMEM)] * 5,
        out_specs=pl.BlockSpec(memory_space=pltpu.VMEM),
        scratch_shapes=[
            pltpu.VMEM((sq, d), jnp.bfloat16),
            pltpu.VMEM((3, sq, d), jnp.bfloat16),
            pltpu.VMEM((d, 3 * d_local), jnp.bfloat16),
            pltpu.VMEM((d_local, d), jnp.bfloat16),
            pltpu.VMEM((sq, d_local), jnp.bfloat16),
            pltpu.VMEM((2, sq, d), jnp.bfloat16),
            pltpu.VMEM((2, hs, d), jnp.bfloat16),
            pltpu.VMEM((2, hs, d), jnp.bfloat16),
            pltpu.VMEM((2, sq, d), jnp.bfloat16),
            pltpu.VMEM((2, hs, d), jnp.bfloat16),
            pltpu.SemaphoreType.DMA((4,)),
            pltpu.SemaphoreType.DMA((4,)),
            pltpu.SemaphoreType.DMA((8,)),
            pltpu.SemaphoreType.DMA((8,)),
        ],
        compiler_params=pltpu.CompilerParams(collective_id=0),
    )(x, Wq, Wo, Wk, Wv)


# device time: 40964 ns/iter; 1.1253x vs baseline; 1.0655x over previous
import jax
import jax.numpy as jnp
from jax import lax
from jax.experimental import pallas as pl
from jax.experimental.pallas import tpu as pltpu

N_DEV = 4
DH = 128
SCALE = 0.08838834764831843


def kernel(x, Wq, Wo, Wk, Wv):
    _, sq, d = x.shape
    d_local = Wq.shape[1]
    n_heads = d_local // DH
    hs = sq // 2

    def body(x_ref, wq_ref, wo_ref, wk_ref, wv_ref, out_ref,
             xloc, xg, wqkv, wob, obuf, pstage, precv,
             xs_sems, xr_sems, s_sems, r_sems):
        my = lax.axis_index("i")
        left = lax.rem(my + (N_DEV - 1), N_DEV)
        right = lax.rem(my + 1, N_DEV)
        diag = lax.rem(my + 2, N_DEV)

        barrier_sem = pltpu.get_barrier_semaphore()
        for nbr in (left, right, diag):
            pl.semaphore_signal(barrier_sem, inc=1, device_id=(nbr,),
                                device_id_type=pl.DeviceIdType.MESH)
        pl.semaphore_wait(barrier_sem, 3)

        def rdma(src, dst, sem_s, sem_r, idx, dev):
            return pltpu.make_async_remote_copy(
                src_ref=src, dst_ref=dst,
                send_sem=sem_s.at[idx], recv_sem=sem_r.at[idx],
                device_id=(dev,), device_id_type=pl.DeviceIdType.MESH,
            )

        xloc[...] = x_ref[0].astype(jnp.bfloat16)

        a_r = rdma(xloc, xg.at[0], xs_sems, xr_sems, 0, right)
        a_l = rdma(xloc, xg.at[1], xs_sems, xr_sems, 1, left)
        a_x = rdma(xloc, xg.at[2], xs_sems, xr_sems, 2, diag)
        r_r = rdma(pstage.at[0], precv.at[0], s_sems, r_sems, 0, right)
        r_l = rdma(pstage.at[1], precv.at[1], s_sems, r_sems, 1, left)
        r_x = rdma(pstage.at[2], precv.at[2], s_sems, r_sems, 2, diag)

        a_r.start()
        a_l.start()
        a_x.start()

        wqkv[:, pl.ds(0, d_local)] = (wq_ref[...] * SCALE).astype(jnp.bfloat16)
        wqkv[:, pl.ds(d_local, d_local)] = wk_ref[...].astype(jnp.bfloat16)
        wqkv[:, pl.ds(2 * d_local, d_local)] = wv_ref[...].astype(jnp.bfloat16)
        wob[...] = wo_ref[...].astype(jnp.bfloat16)

        def attn_rows(qkv, r0):
            for hh in range(n_heads):
                q = qkv[r0:r0 + hs, hh * DH:(hh + 1) * DH]
                k = qkv[:, d_local + hh * DH:d_local + (hh + 1) * DH]
                v = qkv[:, 2 * d_local + hh * DH:2 * d_local + (hh + 1) * DH]
                s = lax.dot_general(
                    q, k, (((1,), (1,)), ((), ())),
                    preferred_element_type=jnp.float32,
                )
                p = jnp.exp(s)
                l = jnp.sum(p, axis=1, keepdims=True)
                o = jnp.dot(p.astype(jnp.bfloat16), v,
                            preferred_element_type=jnp.float32) / l
                obuf[r0:r0 + hs, hh * DH:(hh + 1) * DH] = o.astype(jnp.bfloat16)
            return jnp.dot(obuf[r0:r0 + hs], wob[...],
                           preferred_element_type=jnp.float32)

        def store_partial(xb, slot):
            qkv = jnp.dot(xb, wqkv[...],
                          preferred_element_type=jnp.float32
                          ).astype(jnp.bfloat16)
            pstage[slot, pl.ds(0, hs)] = attn_rows(qkv, 0).astype(jnp.bfloat16)
            pstage[slot, pl.ds(hs, hs)] = attn_rows(qkv, hs).astype(jnp.bfloat16)

        qkv0 = jnp.dot(xloc[...], wqkv[...],
                       preferred_element_type=jnp.float32
                       ).astype(jnp.bfloat16)
        p0t = attn_rows(qkv0, 0)

        a_r.wait_recv()
        store_partial(xg[0], 1)
        r_l.start()

        a_l.wait_recv()
        store_partial(xg[1], 0)
        r_r.start()

        a_x.wait_recv()
        store_partial(xg[2], 2)
        r_x.start()

        p0b = attn_rows(qkv0, hs)

        top = pl.ds(0, hs)
        bot = pl.ds(hs, hs)
        r_r.wait_recv()
        r_l.wait_recv()
        tmp_t = (p0t + precv[0, top].astype(jnp.float32)
                 + precv[1, top].astype(jnp.float32))
        tmp_b = (p0b + precv[0, bot].astype(jnp.float32)
                 + precv[1, bot].astype(jnp.float32))
        r_x.wait_recv()
        out_ref[0, top] = tmp_t + precv[2, top].astype(jnp.float32)
        out_ref[0, bot] = tmp_b + precv[2, bot].astype(jnp.float32)

        for r in (a_r, a_l, a_x, r_r, r_l, r_x):
            r.wait_send()

    return pl.pallas_call(
        body,
        out_shape=jax.ShapeDtypeStruct((1, sq, d), jnp.float32),
        in_specs=[pl.BlockSpec(memory_space=pltpu.VMEM)] * 5,
        out_specs=pl.BlockSpec(memory_space=pltpu.VMEM),
        scratch_shapes=[
            pltpu.VMEM((sq, d), jnp.bfloat16),
            pltpu.VMEM((3, sq, d), jnp.bfloat16),
            pltpu.VMEM((d, 3 * d_local), jnp.bfloat16),
            pltpu.VMEM((d_local, d), jnp.bfloat16),
            pltpu.VMEM((sq, d_local), jnp.bfloat16),
            pltpu.VMEM((3, sq, d), jnp.bfloat16),
            pltpu.VMEM((3, sq, d), jnp.bfloat16),
            pltpu.SemaphoreType.DMA((3,)),
            pltpu.SemaphoreType.DMA((3,)),
            pltpu.SemaphoreType.DMA((3,)),
            pltpu.SemaphoreType.DMA((3,)),
        ],
        compiler_params=pltpu.CompilerParams(collective_id=0),
    )(x, Wq, Wo, Wk, Wv)
